# Initial kernel scaffold; baseline (speedup 1.0000x reference)
import jax
import jax.numpy as jnp
from jax.experimental import pallas as pl


def kernel(x, targets, seen):
    raise NotImplementedError("write your pallas kernel here")


# speedup vs baseline: 4.3083x; 4.3083x over previous
import jax
import jax.numpy as jnp
from jax.experimental import pallas as pl
from jax.experimental.pallas import tpu as pltpu

_B = 64
_G = 13
_GG = _G * _G
_A = 5
_C = 20
_CH = _A * (_C + 5)
_ANCHORS = ((1.3221, 1.73145), (3.19275, 4.00944), (5.05587, 8.09892),
            (9.47112, 4.84053), (11.2364, 10.0071))
_OBJECT_SCALE = 5.0
_NOOBJECT_SCALE = 1.0
_CLASS_SCALE = 1.0
_COORD_SCALE = 1.0
_THRESH = 0.6
_EPS = 1e-16


def _body(xt_ref, tgt_ref, ts_ref, gate_ref, out_ref):
    f32 = jnp.float32

    tgt = tgt_ref[:, :]
    cls_t = tgt[:, 1:2]
    cx = tgt[:, 2:3] * _G
    cy = tgt[:, 3:4] * _G
    twg = tgt[:, 4:5] * _G
    thg = tgt[:, 5:6] * _G
    gxf = jnp.floor(cx)
    gyf = jnp.floor(cy)
    pcol = (gyf * _G + gxf).astype(jnp.int32)

    best = jnp.full((_B, 1), -1.0, f32)
    acol = jnp.zeros((_B, 1), jnp.int32)
    awb = jnp.full((_B, 1), _ANCHORS[0][0], f32)
    ahb = jnp.full((_B, 1), _ANCHORS[0][1], f32)
    for a, (aw, ah) in enumerate(_ANCHORS):
        inter = jnp.minimum(aw, twg) * jnp.minimum(ah, thg)
        union = aw * ah + twg * thg - inter
        r = inter / (union + _EPS)
        upd = r > best
        best = jnp.where(upd, r, best)
        acol = jnp.where(upd, a, acol)
        awb = jnp.where(upd, aw, awb)
        ahb = jnp.where(upd, ah, ahb)

    gi = jax.lax.broadcasted_iota(jnp.int32, (1, _B, _GG), 2)
    msk = (gi == pcol.reshape(1, _B, 1)).astype(f32)
    w = jnp.sum(xt_ref[:, :, :] * msk, axis=2)
    wt = w.T

    chi = jax.lax.broadcasted_iota(jnp.int32, (_B, _CH), 1)
    base = acol * (_C + 5)
    sel = []
    for c in range(_C + 5):
        m = (chi == base + c).astype(f32)
        sel.append(jnp.sum(wt * m, axis=1, keepdims=True))

    txs, tys, tws, ths, cfs = sel[0], sel[1], sel[2], sel[3], sel[4]

    acc_cls = jnp.float32(0.0)
    for j in range(_C):
        pj = jax.nn.sigmoid(sel[5 + j])
        oh = (cls_t == float(j)).astype(f32)
        acc_cls = acc_cls + jnp.sum((pj - oh) ** 2)
    loss_cls = acc_cls / (_B * _C) * _CLASS_SCALE

    sx_t = jax.nn.sigmoid(txs)
    sy_t = jax.nn.sigmoid(tys)
    px = sx_t + gxf
    py = sy_t + gyf
    pw = jnp.exp(tws) * awb
    ph = jnp.exp(ths) * ahb

    ix1 = jnp.maximum(px - pw * 0.5, cx - twg * 0.5)
    ix2 = jnp.minimum(px + pw * 0.5, cx + twg * 0.5)
    iy1 = jnp.maximum(py - ph * 0.5, cy - thg * 0.5)
    iy2 = jnp.minimum(py + ph * 0.5, cy + thg * 0.5)
    iw = jnp.clip(ix2 - ix1, 0.0, None)
    ih = jnp.clip(iy2 - iy1, 0.0, None)
    inter_t = iw * ih
    iou_t = inter_t / (pw * ph + twg * thg - inter_t + _EPS)

    loss_obj = jnp.sum((jax.nn.sigmoid(cfs) - iou_t) ** 2) / _B * _OBJECT_SCALE

    scale = jnp.sqrt(2.0 - twg * thg * (1.0 / (_G * _G)))
    d0 = (sx_t - (cx - gxf)) * scale
    d1 = (sy_t - (cy - gyf)) * scale
    d2 = (tws - jnp.log(twg / awb)) * scale
    d3 = (ths - jnp.log(thg / ahb)) * scale
    loss_coords = (jnp.sum(d0 * d0) + jnp.sum(d1 * d1) + jnp.sum(d2 * d2)
                   + jnp.sum(d3 * d3)) / (_B * 4) * _COORD_SCALE

    gl = jax.lax.broadcasted_iota(jnp.int32, (1, _GG), 1)
    gxg = (gl % _G).astype(f32)
    gyg = (gl // _G).astype(f32)

    acc_nn = jnp.float32(0.0)
    acc_nc = jnp.float32(0.0)
    acc_pr = jnp.float32(0.0)
    acc_pc = jnp.float32(0.0)
    for a, (aw, ah) in enumerate(_ANCHORS):
        tx = xt_ref[a * (_C + 5) + 0]
        ty = xt_ref[a * (_C + 5) + 1]
        tw2 = xt_ref[a * (_C + 5) + 2]
        th2 = xt_ref[a * (_C + 5) + 3]
        cf = xt_ref[a * (_C + 5) + 4]
        sx = jax.nn.sigmoid(tx)
        sy = jax.nn.sigmoid(ty)
        pc = jax.nn.sigmoid(cf)
        bx = sx + gxg
        by = sy + gyg
        bw = jnp.exp(tw2) * aw
        bh = jnp.exp(th2) * ah
        bx1 = bx - bw * 0.5
        bx2 = bx + bw * 0.5
        by1 = by - bh * 0.5
        by2 = by + bh * 0.5
        a1t = bw * bh * _THRESH

        def tstep(t, over):
            cxs = ts_ref[t, 2] * _G
            cys = ts_ref[t, 3] * _G
            tws_ = ts_ref[t, 4] * _G
            ths_ = ts_ref[t, 5] * _G
            tx1 = cxs - tws_ * 0.5
            tx2 = cxs + tws_ * 0.5
            ty1 = cys - ths_ * 0.5
            ty2 = cys + ths_ * 0.5
            rhs = _THRESH * (tws_ * ths_ + _EPS)
            iw_ = jnp.maximum(jnp.minimum(bx2, tx2) - jnp.maximum(bx1, tx1), 0.0)
            ih_ = jnp.maximum(jnp.minimum(by2, ty2) - jnp.maximum(by1, ty1), 0.0)
            it = iw_ * ih_
            hit = it * (1.0 + _THRESH) > a1t + rhs
            return jnp.maximum(over, hit.astype(f32))

        over = jax.lax.fori_loop(0, _B, tstep, jnp.zeros((_B, _GG), f32))

        m_a = jnp.logical_and(acol == a, pcol == gl).astype(f32)
        noobj = (1.0 - m_a) * (1.0 - over)
        acc_nn = acc_nn + jnp.sum(pc * pc * noobj)
        acc_nc = acc_nc + jnp.sum(noobj)
        notm = 1.0 - m_a
        pr = ((sx - 0.5) ** 2 + (sy - 0.5) ** 2 + tw2 * tw2 + th2 * th2)
        acc_pr = acc_pr + jnp.sum(pr * notm)
        acc_pc = acc_pc + jnp.sum(notm)

    loss_noobj = acc_nn / jnp.maximum(acc_nc, 1.0) * _NOOBJECT_SCALE
    loss_prior = acc_pr / jnp.maximum(acc_pc * 4.0, 1.0) * gate_ref[0, 0]

    total = loss_cls + loss_obj + loss_coords + loss_noobj + loss_prior
    out_ref[:, :] = jnp.full((1, 1), total, f32)


def kernel(x, targets, seen):
    xt = jnp.transpose(x.reshape(_B, _CH, _GG), (1, 0, 2))
    gate = jnp.where(jnp.asarray(seen) < 12800, 0.01, 0.0)
    gate = gate.astype(jnp.float32).reshape(1, 1)
    out = pl.pallas_call(
        _body,
        out_shape=jax.ShapeDtypeStruct((1, 1), jnp.float32),
        in_specs=[
            pl.BlockSpec(memory_space=pltpu.VMEM),
            pl.BlockSpec(memory_space=pltpu.VMEM),
            pl.BlockSpec(memory_space=pltpu.SMEM),
            pl.BlockSpec(memory_space=pltpu.SMEM),
        ],
        out_specs=pl.BlockSpec(memory_space=pltpu.VMEM),
    )(xt, targets, targets, gate)
    return out.reshape(())
